# baseline (speedup 1.0000x reference)
import functools

import jax
import jax.numpy as jnp
from jax import lax
from jax.experimental import pallas as pl
from jax.experimental.pallas import tpu as pltpu
from jax.experimental.pallas import tpu_sc as plsc

R = 128
N = 32768
K_SEL = 32
L = 16
NC = 2
NS = 16
NW = NC * NS
ROWS_PW = R // NW
CHUNK = 8192
NCHUNK = N // CHUNK
VPB = 16
VPG = 4
NBLK = CHUNK // (L * VPB)

_GDN = lax.GatherDimensionNumbers(
    offset_dims=(), collapsed_slice_dims=(0,), start_index_map=(0,)
)


def _lane():
    return lax.iota(jnp.int32, L)


def _shuffle(v, j):
    idx = (_lane() ^ j)[:, None]
    return lax.gather(
        v, idx, _GDN, (1,), mode=lax.GatherScatterMode.PROMISE_IN_BOUNDS
    )


def _ce(v, j, take_min):
    pv = _shuffle(v, j)
    return jnp.where(take_min, jnp.minimum(v, pv), jnp.maximum(v, pv))


def _sort16(v):
    lane = _lane()
    sp = 1
    for p in (2, 4, 8, 16):
        j = p // 2
        sj = sp - 1
        while j >= 1:
            take_min = (((lane >> sj) ^ (lane >> sp)) & 1) == 0
            v = _ce(v, j, take_min)
            j //= 2
            sj -= 1
        sp += 1
    return v


def _bitonic_merge16(v):
    lane = _lane()
    for j in (8, 4, 2, 1):
        v = _ce(v, j, (lane & j) == 0)
    return v


def _lane_max(v):
    for j in (8, 4, 2, 1):
        v = jnp.maximum(v, _shuffle(v, j))
    return v


def _lane_sum(v):
    for j in (8, 4, 2, 1):
        v = v + _shuffle(v, j)
    return v


def _merge_body(a0, a1, rv):
    mlo = jnp.maximum(a0, rv)
    n0 = jnp.minimum(mlo, a1)
    n1 = jnp.maximum(mlo, a1)
    return _bitonic_merge16(n0), _bitonic_merge16(n1)


def _merge_topk(a0, a1, v):
    rv = lax.rev(_sort16(v), (0,))
    return _merge_body(a0, a1, rv)


_sc_mesh = plsc.VectorSubcoreMesh(core_axis_name="c", subcore_axis_name="s")


@functools.partial(
    pl.kernel,
    out_type=jax.ShapeDtypeStruct((NW * L,), jnp.float32),
    mesh=_sc_mesh,
    scratch_types=[
        pltpu.VMEM((CHUNK,), jnp.float32),
        pltpu.VMEM((CHUNK,), jnp.float32),
        pltpu.VMEM((2 * L,), jnp.float32),
        pltpu.VMEM((L,), jnp.float32),
        pltpu.SemaphoreType.DMA,
        pltpu.SemaphoreType.DMA,
    ],
)
def _topk_mean_sc(x_hbm, out_hbm, buf0, buf1, topv, means_v, sem0, sem1):
    cid = lax.axis_index("c")
    sid = lax.axis_index("s")
    wid = sid * NC + cid

    means_v[...] = jnp.zeros((L,), jnp.float32)
    bufs = (buf0, buf1)
    sems = (sem0, sem1)

    def _reload_thr(_):
        return topv[pl.ds(0, L)][0]

    def _tree_max(buf, base, n):
        vs = [buf[pl.ds(base + j * L, L)] for j in range(n)]
        while len(vs) > 1:
            vs = [
                jnp.maximum(vs[i], vs[i + 1]) if i + 1 < len(vs) else vs[i]
                for i in range(0, len(vs), 2)
            ]
        return vs[0]

    def _merge_vreg(v, thr_i):
        lane = _lane()
        thr_splat = jnp.full((L,), thr_i, jnp.float32)
        cntf = _lane_sum(
            jnp.where(v > thr_splat, jnp.float32(1.0), jnp.float32(0.0))
        )[0]
        single = cntf == jnp.float32(1.0)

        @pl.when(single)
        def _fast():
            c = jnp.full((L,), _lane_max(v)[0], jnp.float32)
            rv = jnp.where(lane == 0, c, jnp.full((L,), -jnp.inf, jnp.float32))
            a0 = topv[pl.ds(0, L)]
            a1 = topv[pl.ds(L, L)]
            n0, n1 = _merge_body(a0, a1, rv)
            topv[pl.ds(0, L)] = n0
            topv[pl.ds(L, L)] = n1

        @pl.when(jnp.logical_not(single))
        def _general():
            a0 = topv[pl.ds(0, L)]
            a1 = topv[pl.ds(L, L)]
            n0, n1 = _merge_topk(a0, a1, v)
            topv[pl.ds(0, L)] = n0
            topv[pl.ds(L, L)] = n1

    def _run_chunk(buf, thr0):
        def blk_fn(b, thr):
            base = b * (VPB * L)
            m = _tree_max(buf, base, VPB)
            hit = _lane_max(m)[0] > thr

            def _process(thr_in):
                for g in range(VPB // VPG):
                    gbase = base + g * VPG * L
                    gm = _tree_max(buf, gbase, VPG)
                    ghit = _lane_max(gm)[0] > thr_in

                    @pl.when(ghit)
                    def _scan_group(gbase=gbase):
                        def vreg_fn(j, thr_i):
                            v = buf[pl.ds(gbase + j * L, L)]
                            hit_v = _lane_max(v)[0] > thr_i

                            @pl.when(hit_v)
                            def _do_merge():
                                _merge_vreg(v, thr_i)

                            return lax.cond(
                                hit_v, _reload_thr, lambda t: t, thr_i
                            )

                        lax.fori_loop(0, VPG, vreg_fn, thr_in)

                    thr_in = lax.cond(ghit, _reload_thr, lambda t: t, thr_in)
                return thr_in

            return lax.cond(hit, _process, lambda t: t, thr)

        return lax.fori_loop(0, NBLK, blk_fn, thr0)

    def row_fn(r, carry):
        rowbase = (wid * ROWS_PW + r) * N
        neg = jnp.full((L,), -jnp.inf, jnp.float32)
        topv[pl.ds(0, L)] = neg
        topv[pl.ds(L, L)] = neg

        copies = [None] * NCHUNK
        copies[0] = pltpu.async_copy(
            x_hbm.at[pl.ds(rowbase, CHUNK)], bufs[0], sems[0]
        )
        thr = jnp.float32(-jnp.inf)
        for c in range(NCHUNK):
            copies[c].wait()
            if c + 1 < NCHUNK:
                copies[c + 1] = pltpu.async_copy(
                    x_hbm.at[pl.ds(rowbase + (c + 1) * CHUNK, CHUNK)],
                    bufs[(c + 1) % 2],
                    sems[(c + 1) % 2],
                )
            thr = _run_chunk(bufs[c % 2], thr)

        a0 = topv[pl.ds(0, L)]
        a1 = topv[pl.ds(L, L)]
        mean = _lane_sum(a0 + a1) * jnp.float32(1.0 / K_SEL)
        means_v[...] = jnp.where(_lane() == r, mean, means_v[...])
        return carry

    lax.fori_loop(0, ROWS_PW, row_fn, 0)
    pltpu.sync_copy(means_v, out_hbm.at[pl.ds(wid * L, L)])


def kernel(x):
    out = _topk_mean_sc(x.reshape(R * N))
    return out.reshape(NW, L)[:, :ROWS_PW].reshape(R)


# speedup vs baseline: 1.0485x; 1.0485x over previous
import functools

import jax
import jax.numpy as jnp
from jax import lax
from jax.experimental import pallas as pl
from jax.experimental.pallas import tpu as pltpu
from jax.experimental.pallas import tpu_sc as plsc

R = 128
N = 32768
K_SEL = 32
L = 16
NC = 2
NS = 16
NW = NC * NS
ROWS_PW = R // NW
CHUNK = 8192
NCHUNK = N // CHUNK
VPB = 8
NBLK = CHUNK // (L * VPB)

_GDN = lax.GatherDimensionNumbers(
    offset_dims=(), collapsed_slice_dims=(0,), start_index_map=(0,)
)


def _lane():
    return lax.iota(jnp.int32, L)


def _shuffle(v, j):
    idx = (_lane() ^ j)[:, None]
    return lax.gather(
        v, idx, _GDN, (1,), mode=lax.GatherScatterMode.PROMISE_IN_BOUNDS
    )


def _ce(v, j, take_min):
    pv = _shuffle(v, j)
    return jnp.where(take_min, jnp.minimum(v, pv), jnp.maximum(v, pv))


def _sort16(v):
    lane = _lane()
    sp = 1
    for p in (2, 4, 8, 16):
        j = p // 2
        sj = sp - 1
        while j >= 1:
            take_min = (((lane >> sj) ^ (lane >> sp)) & 1) == 0
            v = _ce(v, j, take_min)
            j //= 2
            sj -= 1
        sp += 1
    return v


def _bitonic_merge16(v):
    lane = _lane()
    for j in (8, 4, 2, 1):
        v = _ce(v, j, (lane & j) == 0)
    return v


def _lane_max(v):
    for j in (8, 4, 2, 1):
        v = jnp.maximum(v, _shuffle(v, j))
    return v


def _lane_sum(v):
    for j in (8, 4, 2, 1):
        v = v + _shuffle(v, j)
    return v


def _merge_body(a0, a1, rv):
    mlo = jnp.maximum(a0, rv)
    n0 = jnp.minimum(mlo, a1)
    n1 = jnp.maximum(mlo, a1)
    return _bitonic_merge16(n0), _bitonic_merge16(n1)


def _merge_topk(a0, a1, v):
    rv = lax.rev(_sort16(v), (0,))
    return _merge_body(a0, a1, rv)


_sc_mesh = plsc.VectorSubcoreMesh(core_axis_name="c", subcore_axis_name="s")


@functools.partial(
    pl.kernel,
    out_type=jax.ShapeDtypeStruct((NW * L,), jnp.float32),
    mesh=_sc_mesh,
    scratch_types=[
        pltpu.VMEM((CHUNK,), jnp.float32),
        pltpu.VMEM((CHUNK,), jnp.float32),
        pltpu.VMEM((2 * L,), jnp.float32),
        pltpu.VMEM((L,), jnp.float32),
        pltpu.SemaphoreType.DMA,
        pltpu.SemaphoreType.DMA,
    ],
)
def _topk_mean_sc(x_hbm, out_hbm, buf0, buf1, topv, means_v, sem0, sem1):
    cid = lax.axis_index("c")
    sid = lax.axis_index("s")
    wid = sid * NC + cid

    means_v[...] = jnp.zeros((L,), jnp.float32)
    bufs = (buf0, buf1)
    sems = (sem0, sem1)

    def _tree_max(buf, base, n):
        vs = [buf[pl.ds(base + j * L, L)] for j in range(n)]
        while len(vs) > 1:
            vs = [
                jnp.maximum(vs[i], vs[i + 1]) if i + 1 < len(vs) else vs[i]
                for i in range(0, len(vs), 2)
            ]
        return vs[0]

    def _run_chunk(buf, thr0):
        def blk_fn(b, thr):
            base = b * (VPB * L)
            m = _tree_max(buf, base, VPB)
            hit = _lane_max(m)[0] > thr

            def _process(thr_in):
                def vreg_fn(j, thr_i):
                    v = buf[pl.ds(base + j * L, L)]
                    hit_v = _lane_max(v)[0] > thr_i

                    def _merge_and_thr(t):
                        del t
                        a0 = topv[pl.ds(0, L)]
                        a1 = topv[pl.ds(L, L)]
                        n0, n1 = _merge_topk(a0, a1, v)
                        topv[pl.ds(0, L)] = n0
                        topv[pl.ds(L, L)] = n1
                        return n0[0]

                    return lax.cond(hit_v, _merge_and_thr, lambda t: t, thr_i)

                return lax.fori_loop(0, VPB, vreg_fn, thr_in)

            return lax.cond(hit, _process, lambda t: t, thr)

        return lax.fori_loop(0, NBLK, blk_fn, thr0)

    def row_fn(r, carry):
        rowbase = (wid * ROWS_PW + r) * N
        neg = jnp.full((L,), -jnp.inf, jnp.float32)
        topv[pl.ds(0, L)] = neg
        topv[pl.ds(L, L)] = neg

        copies = [None] * NCHUNK
        copies[0] = pltpu.async_copy(
            x_hbm.at[pl.ds(rowbase, CHUNK)], bufs[0], sems[0]
        )
        thr = jnp.float32(-jnp.inf)
        for c in range(NCHUNK):
            copies[c].wait()
            if c + 1 < NCHUNK:
                copies[c + 1] = pltpu.async_copy(
                    x_hbm.at[pl.ds(rowbase + (c + 1) * CHUNK, CHUNK)],
                    bufs[(c + 1) % 2],
                    sems[(c + 1) % 2],
                )
            thr = _run_chunk(bufs[c % 2], thr)

        a0 = topv[pl.ds(0, L)]
        a1 = topv[pl.ds(L, L)]
        mean = _lane_sum(a0 + a1) * jnp.float32(1.0 / K_SEL)
        means_v[...] = jnp.where(_lane() == r, mean, means_v[...])
        return carry

    lax.fori_loop(0, ROWS_PW, row_fn, 0)
    pltpu.sync_copy(means_v, out_hbm.at[pl.ds(wid * L, L)])


def kernel(x):
    out = _topk_mean_sc(x.reshape(R * N))
    return out.reshape(NW, L)[:, :ROWS_PW].reshape(R)


# speedup vs baseline: 1.4610x; 1.3935x over previous
import functools

import jax
import jax.numpy as jnp
from jax import lax
from jax.experimental import pallas as pl
from jax.experimental.pallas import tpu as pltpu
from jax.experimental.pallas import tpu_sc as plsc

R = 128
N = 32768
K_SEL = 32
L = 16
NC = 2
NS = 16
NW = NC * NS
ROWS_PW = R // NW
CHUNK = 8192
NCHUNK = N // CHUNK
VPB = 8
NBLK = CHUNK // (L * VPB)

_GDN = lax.GatherDimensionNumbers(
    offset_dims=(), collapsed_slice_dims=(0,), start_index_map=(0,)
)


def _lane():
    return lax.iota(jnp.int32, L)


def _make_idx():
    lane = _lane()
    return {j: (lane ^ j)[:, None] for j in (1, 2, 4, 8)}


def _shuffle(X, v, j):
    return lax.gather(
        v, X[j], _GDN, (1,), mode=lax.GatherScatterMode.PROMISE_IN_BOUNDS
    )


def _ce(X, v, j, take_min):
    pv = _shuffle(X, v, j)
    return jnp.where(take_min, jnp.minimum(v, pv), jnp.maximum(v, pv))


def _sort16(X, v):
    lane = _lane()
    sp = 1
    for p in (2, 4, 8, 16):
        j = p // 2
        sj = sp - 1
        while j >= 1:
            take_min = (((lane >> sj) ^ (lane >> sp)) & 1) == 0
            v = _ce(X, v, j, take_min)
            j //= 2
            sj -= 1
        sp += 1
    return v


def _bitonic_merge16(X, v):
    lane = _lane()
    for j in (8, 4, 2, 1):
        v = _ce(X, v, j, (lane & j) == 0)
    return v


def _lane_max(X, v):
    for j in (8, 4, 2, 1):
        v = jnp.maximum(v, _shuffle(X, v, j))
    return v


def _lane_or(X, v):
    for j in (8, 4, 2, 1):
        v = v | _shuffle(X, v, j)
    return v


def _lane_sum(X, v):
    for j in (8, 4, 2, 1):
        v = v + _shuffle(X, v, j)
    return v


def _merge_topk(X, a0, a1, v):
    rv = lax.rev(_sort16(X, v), (0,))
    mlo = jnp.maximum(a0, rv)
    n0 = jnp.minimum(mlo, a1)
    n1 = jnp.maximum(mlo, a1)
    return _bitonic_merge16(X, n0), _bitonic_merge16(X, n1)


_sc_mesh = plsc.VectorSubcoreMesh(core_axis_name="c", subcore_axis_name="s")


@functools.partial(
    pl.kernel,
    out_type=jax.ShapeDtypeStruct((NW * L,), jnp.float32),
    mesh=_sc_mesh,
    scratch_types=[
        pltpu.VMEM((CHUNK,), jnp.float32),
        pltpu.VMEM((CHUNK,), jnp.float32),
        pltpu.VMEM((2 * L,), jnp.float32),
        pltpu.VMEM((L,), jnp.float32),
        pltpu.SemaphoreType.DMA,
        pltpu.SemaphoreType.DMA,
    ],
)
def _topk_mean_sc(x_hbm, out_hbm, buf0, buf1, topv, means_v, sem0, sem1):
    cid = lax.axis_index("c")
    sid = lax.axis_index("s")
    wid = sid * NC + cid
    X = _make_idx()

    means_v[...] = jnp.zeros((L,), jnp.float32)
    bufs = (buf0, buf1)
    sems = (sem0, sem1)

    def _run_chunk(buf, thr0):
        def blk_fn(b, thr):
            base = b * (VPB * L)
            vs = [buf[pl.ds(base + j * L, L)] for j in range(VPB)]
            t = vs
            while len(t) > 1:
                t = [jnp.maximum(t[i], t[i + 1]) for i in range(0, len(t), 2)]
            hit = _lane_max(X, t[0])[0] > thr

            def _process(thr_in):
                thr_splat = jnp.full((L,), thr_in, jnp.float32)
                bmv = jnp.zeros((L,), jnp.int32)
                for j in range(VPB):
                    v = buf[pl.ds(base + j * L, L)]
                    bmv = bmv | jnp.where(
                        v > thr_splat, jnp.int32(1 << j), jnp.int32(0)
                    )
                bm = _lane_or(X, bmv)[0]

                thr_cur = thr_in
                for j in range(VPB):
                    def _merge_j(t, j=j):
                        del t
                        v = buf[pl.ds(base + j * L, L)]
                        a0 = topv[pl.ds(0, L)]
                        a1 = topv[pl.ds(L, L)]
                        n0, n1 = _merge_topk(X, a0, a1, v)
                        topv[pl.ds(0, L)] = n0
                        topv[pl.ds(L, L)] = n1
                        return n0[0]

                    flagged = ((bm >> j) & 1) != 0
                    thr_cur = lax.cond(flagged, _merge_j, lambda t: t, thr_cur)
                return thr_cur

            return lax.cond(hit, _process, lambda t: t, thr)

        return lax.fori_loop(0, NBLK, blk_fn, thr0)

    def row_fn(r, carry):
        rowbase = (wid * ROWS_PW + r) * N
        neg = jnp.full((L,), -jnp.inf, jnp.float32)
        topv[pl.ds(0, L)] = neg
        topv[pl.ds(L, L)] = neg

        copies = [None] * NCHUNK
        copies[0] = pltpu.async_copy(
            x_hbm.at[pl.ds(rowbase, CHUNK)], bufs[0], sems[0]
        )
        thr = jnp.float32(-jnp.inf)
        for c in range(NCHUNK):
            copies[c].wait()
            if c + 1 < NCHUNK:
                copies[c + 1] = pltpu.async_copy(
                    x_hbm.at[pl.ds(rowbase + (c + 1) * CHUNK, CHUNK)],
                    bufs[(c + 1) % 2],
                    sems[(c + 1) % 2],
                )
            thr = _run_chunk(bufs[c % 2], thr)

        a0 = topv[pl.ds(0, L)]
        a1 = topv[pl.ds(L, L)]
        mean = _lane_sum(X, a0 + a1) * jnp.float32(1.0 / K_SEL)
        means_v[...] = jnp.where(_lane() == r, mean, means_v[...])
        return carry

    lax.fori_loop(0, ROWS_PW, row_fn, 0)
    pltpu.sync_copy(means_v, out_hbm.at[pl.ds(wid * L, L)])


def kernel(x):
    out = _topk_mean_sc(x.reshape(R * N))
    return out.reshape(NW, L)[:, :ROWS_PW].reshape(R)
